# baseline (speedup 1.0000x reference)
import functools

import jax
import jax.numpy as jnp
from jax import lax
from jax.experimental import pallas as pl
from jax.experimental.pallas import tpu as pltpu
from jax.experimental.pallas import tpu_sc as plsc

N = 10000
E = 160000
P = 256
M = 128
OUT = 40
KAPPA = 0.9
MITR = 8
POWER_ITERS = 30

NC = 2
NS = 16
H = 64
CH = 128
CHUNKS = 80
EPT = CH * CHUNKS
EPAD = EPT * NS
NPAD = 10240
NP = 10240
BN = 2048

_mesh = plsc.VectorSubcoreMesh(
    core_axis_name="c", subcore_axis_name="s", num_cores=NC, num_subcores=NS)


_GDN = lax.GatherDimensionNumbers(
    offset_dims=(), collapsed_slice_dims=(0,), start_index_map=(0,))


def _dyngather16(v, idx):
    return lax.gather(v, idx[:, None], _GDN, slice_sizes=(1,),
                      mode=lax.GatherScatterMode.PROMISE_IN_BOUNDS)


def _vsum16(v):
    idx = lax.iota(jnp.int32, 16)
    for sh in (8, 4, 2, 1):
        v = v + _dyngather16(v, idx ^ sh)
    return v


def _power_body(rc_hbm, vals_hbm, r2_hbm,
                rcv, valv, rowi, coli, vbuf, gath, prod, zv, obuf,
                wa, wb, gsem, ssem):
    c = lax.axis_index("c")
    s = lax.axis_index("s")
    stripe = NPAD // NS

    pltpu.sync_copy(rc_hbm.at[pl.ds(s * CHUNKS, CHUNKS)], rcv)
    pltpu.sync_copy(vals_hbm.at[pl.ds(s * CHUNKS, CHUNKS)], valv)

    def unpack(j, _):
        for q in range(CH // 16):
            rc16 = rcv[j, pl.ds(q * 16, 16)]
            rowi[j, pl.ds(q * 16, 16)] = lax.shift_right_logical(rc16, 14)
            coli[j, pl.ds(q * 16, 16)] = lax.bitwise_and(rc16, 16383)
        return 0
    lax.fori_loop(0, CHUNKS, unpack, 0)

    c001 = jnp.full((16,), 0.01, jnp.float32)
    z16 = jnp.zeros((16,), jnp.float32)

    def init_c(g, _):
        zv[pl.ds(g * 16, 16)] = c001
        return 0
    lax.fori_loop(0, stripe // 16, init_c, 0)
    pltpu.sync_copy(zv, wa.at[pl.ds(s * stripe, stripe)])

    def init_z(g, _):
        zv[pl.ds(g * 16, 16)] = z16
        return 0
    lax.fori_loop(0, stripe // 16, init_z, 0)
    plsc.subcore_barrier()

    def half_iter(wcur, wnxt, carry):
        inv16, ss_prev, ss_cur = carry

        pltpu.sync_copy(zv, wnxt.at[pl.ds(s * stripe, stripe)])
        plsc.subcore_barrier()

        def gissue(j, _):
            pltpu.async_copy(wcur.at[coli.at[j]], gath.at[j], gsem)
            return 0
        lax.fori_loop(0, CHUNKS, gissue, 0)

        def gdrain(j, _):
            pltpu.make_async_copy(
                wcur.at[coli.at[0]], gath.at[0], gsem).wait()
            return 0
        lax.fori_loop(0, CHUNKS, gdrain, 0)

        def pcompute(j, _):
            for q in range(CH // 16):
                prod[j, pl.ds(q * 16, 16)] = \
                    valv[j, pl.ds(q * 16, 16)] * \
                    (gath[j, pl.ds(q * 16, 16)] * inv16)
            pltpu.async_copy(prod.at[j], wnxt.at[rowi.at[j]], ssem, add=True)
            return 0
        lax.fori_loop(0, CHUNKS, pcompute, 0)

        def sdrain(j, _):
            pltpu.make_async_copy(
                prod.at[0], wnxt.at[rowi.at[0]], ssem).wait()
            return 0
        lax.fori_loop(0, CHUNKS, sdrain, 0)
        plsc.subcore_barrier()

        pltpu.sync_copy(wnxt.at[pl.ds(0, N)], vbuf.at[pl.ds(0, N)])

        acc = jnp.zeros((16,), jnp.float32)

        def ssbody(g, a):
            w16 = vbuf[pl.ds(g * 16, 16)]
            return a + w16 * w16
        acc = lax.fori_loop(0, N // 16, ssbody, acc)
        ss = _vsum16(acc)
        plsc.subcore_barrier()
        return (1.0 / ss, ss_cur, ss)

    def iter_pair(_, carry):
        carry = half_iter(wa, wb, carry)
        carry = half_iter(wb, wa, carry)
        return carry

    one = jnp.ones((16,), jnp.float32)
    _, ss_prev, ss_cur = lax.fori_loop(
        0, POWER_ITERS // 2, iter_pair, (one, one, one))

    r2 = ss_prev * ss_cur

    @pl.when(jnp.logical_and(c == 0, s == 0))
    def _():
        obuf[...] = r2
        pltpu.sync_copy(obuf, r2_hbm)


_power_kernel = functools.partial(
    pl.kernel,
    out_type=jax.ShapeDtypeStruct((16,), jnp.float32),
    mesh=_mesh,
    scratch_types=[
        pltpu.VMEM((CHUNKS, CH), jnp.int32),
        pltpu.VMEM((CHUNKS, CH), jnp.float32),
        pltpu.VMEM((CHUNKS, CH), jnp.int32),
        pltpu.VMEM((CHUNKS, CH), jnp.int32),
        pltpu.VMEM((NPAD,), jnp.float32),
        pltpu.VMEM((CHUNKS, CH), jnp.float32),
        pltpu.VMEM((CHUNKS, CH), jnp.float32),
        pltpu.VMEM((NPAD // NS,), jnp.float32),
        pltpu.VMEM((16,), jnp.float32),
        pltpu.VMEM_SHARED((NPAD,), jnp.float32),
        pltpu.VMEM_SHARED((NPAD,), jnp.float32),
        pltpu.SemaphoreType.DMA,
        pltpu.SemaphoreType.DMA,
    ],
)(_power_body)


NBUF = 2
NH = NP // NC
ROWS = NH // NS
SCH = 44
CAP = NS * SCH * CH


def _spmm_body(rc_hbm, vals_hbm, x_hbm, xa_hbm,
               rcbuf, valbuf, mvbuf, rowi, coli, gbuf, zbuf, xash,
               esems, vsems, gsems, ssems):
    c = lax.axis_index("c")
    s = lax.axis_index("s")
    cbase = c * NH

    z16 = jnp.zeros((16,), jnp.float32)

    def zinit(e, _):
        for q in range(M // 16):
            zbuf[e, pl.ds(q * 16, 16)] = z16
        return 0
    lax.fori_loop(0, 64, zinit, 0)
    for k in range(ROWS // 64):
        pltpu.sync_copy(zbuf, xash.at[pl.ds(s * ROWS + k * 64, 64)])

    plsc.subcore_barrier()

    def eissue(i, b):
        jr = (c * NS + s) * SCH + i * NBUF + b
        pltpu.async_copy(rc_hbm.at[jr], rcbuf.at[b], esems.at[b])
        pltpu.async_copy(vals_hbm.at[jr], valbuf.at[b], vsems.at[b])

    def chunk_in(i, b):
        pltpu.make_async_copy(rc_hbm.at[0], rcbuf.at[b], esems.at[b]).wait()
        pltpu.make_async_copy(vals_hbm.at[0], valbuf.at[b], vsems.at[b]).wait()
        for q in range(CH // 16):
            rc16 = rcbuf[b, pl.ds(q * 16, 16)]
            row16 = lax.shift_right_logical(rc16, 14)
            col16 = lax.bitwise_and(rc16, 16383)
            lcol = col16 - cbase
            inhalf = jnp.logical_and(lcol >= 0, lcol < NH)
            rowi[b, pl.ds(q * 16, 16)] = row16
            coli[b, pl.ds(q * 16, 16)] = jnp.clip(lcol, 0, NH - 1)
            mvbuf[b, pl.ds(q * 16, 16)] = jnp.where(
                inhalf, valbuf[b, pl.ds(q * 16, 16)], 0.0)
        pltpu.async_copy(x_hbm.at[rowi.at[b]], gbuf.at[b], gsems.at[b])

    def chunk_out(i, b):
        pltpu.make_async_copy(
            x_hbm.at[rowi.at[b]], gbuf.at[b], gsems.at[b]).wait()

        def mul(g, _):
            val16 = mvbuf[b, pl.ds(g * 16, 16)]
            for k in range(16):
                e = g * 16 + k
                v = val16[k]
                for q in range(M // 16):
                    gbuf[b, e, pl.ds(q * 16, 16)] = \
                        gbuf[b, e, pl.ds(q * 16, 16)] * v
            return 0
        lax.fori_loop(0, CH // 16, mul, 0)
        pltpu.async_copy(
            gbuf.at[b], xash.at[coli.at[b]], ssems.at[b], add=True)

    for b in range(NBUF):
        eissue(0, b)

    def outer(i, _):
        @pl.when(i > 0)
        def _():
            for b in range(NBUF):
                pltpu.make_async_copy(
                    gbuf.at[b], xash.at[coli.at[b]], ssems.at[b]).wait()
        for b in range(NBUF):
            chunk_in(i, b)

        @pl.when(i < SCH // NBUF - 1)
        def _():
            for b in range(NBUF):
                eissue(i + 1, b)
        for b in range(NBUF):
            chunk_out(i, b)
        return 0
    lax.fori_loop(0, SCH // NBUF, outer, 0, unroll=False)

    for b in range(NBUF):
        pltpu.make_async_copy(
            gbuf.at[b], xash.at[coli.at[b]], ssems.at[b]).wait()
    plsc.subcore_barrier()

    pltpu.sync_copy(xash.at[pl.ds(s * ROWS, ROWS)],
                    xa_hbm.at[pl.ds(cbase + s * ROWS, ROWS)])


_spmm_kernel = functools.partial(
    pl.kernel,
    out_type=jax.ShapeDtypeStruct((NP, M), jnp.float32),
    mesh=_mesh,
    scratch_types=[
        pltpu.VMEM((NBUF, CH), jnp.int32),
        pltpu.VMEM((NBUF, CH), jnp.float32),
        pltpu.VMEM((NBUF, CH), jnp.float32),
        pltpu.VMEM((NBUF, CH), jnp.int32),
        pltpu.VMEM((NBUF, CH), jnp.int32),
        pltpu.VMEM((NBUF, CH, M), jnp.float32),
        pltpu.VMEM((64, M), jnp.float32),
        pltpu.VMEM_SHARED((NH, M), jnp.float32),
        pltpu.SemaphoreType.DMA((NBUF,)),
        pltpu.SemaphoreType.DMA((NBUF,)),
        pltpu.SemaphoreType.DMA((NBUF,)),
        pltpu.SemaphoreType.DMA((NBUF,)),
    ],
)(_spmm_body)


def _prep_body(x_ref, om0_ref, om1_ref, w0_ref, w1_ref, r2_ref,
               b0o, b1o, wt0, wt1):
    i = pl.program_id(0)
    xb = x_ref[...]
    dn = (((1,), (1,)), ((), ()))
    b0o[...] = lax.dot_general(xb, om0_ref[...], dn,
                               preferred_element_type=jnp.float32)
    b1o[...] = lax.dot_general(xb, om1_ref[...], dn,
                               preferred_element_type=jnp.float32)

    @pl.when(i == 0)
    def _():
        rho = jnp.sqrt(r2_ref[0, 0])
        kr = KAPPA / rho
        for w_ref, wt in ((w0_ref, wt0), (w1_ref, wt1)):
            w = w_ref[...]
            sa = jnp.sum(jnp.abs(w), axis=1)
            scale = jnp.where(sa > kr, kr / (sa + 1e-12), 1.0)
            wt[...] = (w * scale[:, None]).T


def _prep(x, Om0, Om1, W0, W1, r2):
    return pl.pallas_call(
        _prep_body,
        grid=(NP // BN,),
        in_specs=[
            pl.BlockSpec((BN, P), lambda i: (i, 0)),
            pl.BlockSpec((M, P), lambda i: (0, 0)),
            pl.BlockSpec((M, P), lambda i: (0, 0)),
            pl.BlockSpec((M, M), lambda i: (0, 0)),
            pl.BlockSpec((M, M), lambda i: (0, 0)),
            pl.BlockSpec((1, 16), lambda i: (0, 0)),
        ],
        out_specs=[
            pl.BlockSpec((BN, M), lambda i: (i, 0)),
            pl.BlockSpec((BN, M), lambda i: (i, 0)),
            pl.BlockSpec((M, M), lambda i: (0, 0)),
            pl.BlockSpec((M, M), lambda i: (0, 0)),
        ],
        out_shape=[
            jax.ShapeDtypeStruct((NP, M), jnp.float32),
            jax.ShapeDtypeStruct((NP, M), jnp.float32),
            jax.ShapeDtypeStruct((M, M), jnp.float32),
            jax.ShapeDtypeStruct((M, M), jnp.float32),
        ],
    )(x, Om0, Om1, W0, W1, r2)


def _step_body(xa_ref, wt_ref, b_ref, x_ref):
    acc = jnp.dot(xa_ref[...], wt_ref[...],
                  preferred_element_type=jnp.float32)
    x_ref[...] = jnp.maximum(acc + b_ref[...], 0.0)


def _step(xa, WnT, b):
    return pl.pallas_call(
        _step_body,
        grid=(NP // BN,),
        in_specs=[
            pl.BlockSpec((BN, M), lambda i: (i, 0)),
            pl.BlockSpec((M, M), lambda i: (0, 0)),
            pl.BlockSpec((BN, M), lambda i: (i, 0)),
        ],
        out_specs=pl.BlockSpec((BN, M), lambda i: (i, 0)),
        out_shape=jax.ShapeDtypeStruct((NP, M), jnp.float32),
    )(xa, WnT, b)


def _head_body(x_ref, hw_ref, hb_ref, o_ref):
    acc = jnp.dot(x_ref[...], hw_ref[...],
                  preferred_element_type=jnp.float32)
    o_ref[...] = acc + hb_ref[...]


def _head(x, head_W, head_b):
    return pl.pallas_call(
        _head_body,
        grid=(NP // BN,),
        in_specs=[
            pl.BlockSpec((BN, M), lambda i: (i, 0)),
            pl.BlockSpec((M, OUT), lambda i: (0, 0)),
            pl.BlockSpec((1, OUT), lambda i: (0, 0)),
        ],
        out_specs=pl.BlockSpec((BN, OUT), lambda i: (i, 0)),
        out_shape=jax.ShapeDtypeStruct((NP, OUT), jnp.float32),
    )(x, head_W, head_b)


def kernel(node_index, x, edge_index, adj_values, emb, W0, Om0, W1, Om1,
           head_W, head_b):
    row = edge_index[0].astype(jnp.int32)
    col = edge_index[1].astype(jnp.int32)
    rc = jnp.concatenate([
        (row << 14) | col,
        (jnp.arange(EPAD - E, dtype=jnp.int32) % N) * 16385,
    ]).reshape(NS * CHUNKS, CH)
    vals = jnp.concatenate([
        adj_values, jnp.zeros((EPAD - E,), jnp.float32)
    ]).reshape(NS * CHUNKS, CH)

    r2 = _power_kernel(rc, vals)

    rcflat = (row << 14) | col
    halfmask = (col >= NH).astype(jnp.int32)
    c1 = jnp.cumsum(halfmask)
    c0 = jnp.arange(1, E + 1, dtype=jnp.int32) - c1
    pos = jnp.where(halfmask == 0, c0 - 1, CAP + c1 - 1)
    pad_rc = (jnp.arange(2 * CAP, dtype=jnp.int32) % N) * 16385
    init = jnp.stack([pad_rc, jnp.zeros((2 * CAP,), jnp.int32)], axis=1)
    pairs = jnp.stack(
        [rcflat, lax.bitcast_convert_type(adj_values, jnp.int32)], axis=1)
    part = init.at[pos].set(pairs, unique_indices=True)
    rc_p = part[:, 0].reshape(-1, CH)
    vals_p = lax.bitcast_convert_type(part[:, 1], jnp.float32).reshape(-1, CH)

    xp = jnp.pad(x, ((0, NP - N), (0, 0)))
    B0, B1, Wt0, Wt1 = _prep(xp, Om0, Om1, W0, W1, r2.reshape(1, 16))

    X = emb
    X = jnp.pad(X, ((0, NP - N), (0, 0)))
    for Wt, B in ((Wt0, B0), (Wt1, B1)):
        for _ in range(MITR):
            xa = _spmm_kernel(rc_p, vals_p, X)
            X = _step(xa, Wt, B)
    return _head(X, head_W, head_b.reshape(1, OUT))[:N]


# speedup vs baseline: 1.2741x; 1.2741x over previous
import functools

import jax
import jax.numpy as jnp
from jax import lax
from jax.experimental import pallas as pl
from jax.experimental.pallas import tpu as pltpu
from jax.experimental.pallas import tpu_sc as plsc

N = 10000
E = 160000
P = 256
M = 128
OUT = 40
KAPPA = 0.9
MITR = 8
POWER_ITERS = 30

NC = 2
NS = 16
H = 64
CH = 128
CHUNKS = 80
EPT = CH * CHUNKS
EPAD = EPT * NS
NPAD = 10240
NP = 10240
BN = 2048

_mesh = plsc.VectorSubcoreMesh(
    core_axis_name="c", subcore_axis_name="s", num_cores=NC, num_subcores=NS)


_GDN = lax.GatherDimensionNumbers(
    offset_dims=(), collapsed_slice_dims=(0,), start_index_map=(0,))


def _dyngather16(v, idx):
    return lax.gather(v, idx[:, None], _GDN, slice_sizes=(1,),
                      mode=lax.GatherScatterMode.PROMISE_IN_BOUNDS)


def _vsum16(v):
    idx = lax.iota(jnp.int32, 16)
    for sh in (8, 4, 2, 1):
        v = v + _dyngather16(v, idx ^ sh)
    return v


def _power_body(rc_hbm, vals_hbm, r2_hbm,
                rcv, valv, rowi, coli, vbuf, gath, prod, zv, obuf,
                wa, wb, gsem, ssem):
    c = lax.axis_index("c")
    s = lax.axis_index("s")
    stripe = NPAD // NS

    pltpu.sync_copy(rc_hbm.at[pl.ds(s * CHUNKS, CHUNKS)], rcv)
    pltpu.sync_copy(vals_hbm.at[pl.ds(s * CHUNKS, CHUNKS)], valv)

    def unpack(j, _):
        for q in range(CH // 16):
            rc16 = rcv[j, pl.ds(q * 16, 16)]
            rowi[j, pl.ds(q * 16, 16)] = lax.shift_right_logical(rc16, 14)
            coli[j, pl.ds(q * 16, 16)] = lax.bitwise_and(rc16, 16383)
        return 0
    lax.fori_loop(0, CHUNKS, unpack, 0)

    c001 = jnp.full((16,), 0.01, jnp.float32)
    z16 = jnp.zeros((16,), jnp.float32)

    def init_c(g, _):
        zv[pl.ds(g * 16, 16)] = c001
        return 0
    lax.fori_loop(0, stripe // 16, init_c, 0)
    pltpu.sync_copy(zv, wa.at[pl.ds(s * stripe, stripe)])

    def init_z(g, _):
        zv[pl.ds(g * 16, 16)] = z16
        return 0
    lax.fori_loop(0, stripe // 16, init_z, 0)
    plsc.subcore_barrier()

    def half_iter(wcur, wnxt, carry):
        inv16, ss_prev, ss_cur = carry

        pltpu.sync_copy(zv, wnxt.at[pl.ds(s * stripe, stripe)])
        plsc.subcore_barrier()

        def gissue(j, _):
            pltpu.async_copy(wcur.at[coli.at[j]], gath.at[j], gsem)
            return 0
        lax.fori_loop(0, CHUNKS, gissue, 0)

        def gdrain(j, _):
            pltpu.make_async_copy(
                wcur.at[coli.at[0]], gath.at[0], gsem).wait()
            return 0
        lax.fori_loop(0, CHUNKS, gdrain, 0)

        def pcompute(j, _):
            for q in range(CH // 16):
                prod[j, pl.ds(q * 16, 16)] = \
                    valv[j, pl.ds(q * 16, 16)] * \
                    (gath[j, pl.ds(q * 16, 16)] * inv16)
            pltpu.async_copy(prod.at[j], wnxt.at[rowi.at[j]], ssem, add=True)
            return 0
        lax.fori_loop(0, CHUNKS, pcompute, 0)

        def sdrain(j, _):
            pltpu.make_async_copy(
                prod.at[0], wnxt.at[rowi.at[0]], ssem).wait()
            return 0
        lax.fori_loop(0, CHUNKS, sdrain, 0)
        plsc.subcore_barrier()

        pltpu.sync_copy(wnxt.at[pl.ds(0, N)], vbuf.at[pl.ds(0, N)])

        acc = jnp.zeros((16,), jnp.float32)

        def ssbody(g, a):
            w16 = vbuf[pl.ds(g * 16, 16)]
            return a + w16 * w16
        acc = lax.fori_loop(0, N // 16, ssbody, acc)
        ss = _vsum16(acc)
        plsc.subcore_barrier()
        return (1.0 / ss, ss_cur, ss)

    def iter_pair(_, carry):
        carry = half_iter(wa, wb, carry)
        carry = half_iter(wb, wa, carry)
        return carry

    one = jnp.ones((16,), jnp.float32)
    _, ss_prev, ss_cur = lax.fori_loop(
        0, POWER_ITERS // 2, iter_pair, (one, one, one))

    r2 = ss_prev * ss_cur

    @pl.when(jnp.logical_and(c == 0, s == 0))
    def _():
        obuf[...] = r2
        pltpu.sync_copy(obuf, r2_hbm)


_power_kernel = functools.partial(
    pl.kernel,
    out_type=jax.ShapeDtypeStruct((16,), jnp.float32),
    mesh=_mesh,
    scratch_types=[
        pltpu.VMEM((CHUNKS, CH), jnp.int32),
        pltpu.VMEM((CHUNKS, CH), jnp.float32),
        pltpu.VMEM((CHUNKS, CH), jnp.int32),
        pltpu.VMEM((CHUNKS, CH), jnp.int32),
        pltpu.VMEM((NPAD,), jnp.float32),
        pltpu.VMEM((CHUNKS, CH), jnp.float32),
        pltpu.VMEM((CHUNKS, CH), jnp.float32),
        pltpu.VMEM((NPAD // NS,), jnp.float32),
        pltpu.VMEM((16,), jnp.float32),
        pltpu.VMEM_SHARED((NPAD,), jnp.float32),
        pltpu.VMEM_SHARED((NPAD,), jnp.float32),
        pltpu.SemaphoreType.DMA,
        pltpu.SemaphoreType.DMA,
    ],
)(_power_body)


NBUF = 2
NH = NP // NC
ROWS = NH // NS
SCH = 44
CAP = NS * SCH * CH


def _spmm_body(rc_hbm, vals_hbm, x_hbm, xa_hbm,
               rcbuf, valbuf, mvbuf, rowi, coli, gbuf, zbuf, xash,
               esems, vsems, gsems, ssems):
    c = lax.axis_index("c")
    s = lax.axis_index("s")
    cbase = c * NH

    z16 = jnp.zeros((16,), jnp.float32)

    def zinit(e, _):
        for q in range(M // 16):
            zbuf[e, pl.ds(q * 16, 16)] = z16
        return 0
    lax.fori_loop(0, 64, zinit, 0)
    for k in range(ROWS // 64):
        pltpu.sync_copy(zbuf, xash.at[pl.ds(s * ROWS + k * 64, 64)])

    plsc.subcore_barrier()

    def eissue(i, b):
        jr = (c * NS + s) * SCH + i * NBUF + b
        pltpu.async_copy(rc_hbm.at[jr], rcbuf.at[b], esems.at[b])
        pltpu.async_copy(vals_hbm.at[jr], valbuf.at[b], vsems.at[b])

    def chunk_in(i, b):
        pltpu.make_async_copy(rc_hbm.at[0], rcbuf.at[b], esems.at[b]).wait()
        pltpu.make_async_copy(vals_hbm.at[0], valbuf.at[b], vsems.at[b]).wait()
        for q in range(CH // 16):
            rc16 = rcbuf[b, pl.ds(q * 16, 16)]
            row16 = lax.shift_right_logical(rc16, 14)
            col16 = lax.bitwise_and(rc16, 16383)
            lcol = col16 - cbase
            inhalf = jnp.logical_and(lcol >= 0, lcol < NH)
            rowi[b, pl.ds(q * 16, 16)] = row16
            coli[b, pl.ds(q * 16, 16)] = jnp.clip(lcol, 0, NH - 1)
            mvbuf[b, pl.ds(q * 16, 16)] = jnp.where(
                inhalf, valbuf[b, pl.ds(q * 16, 16)], 0.0)
        pltpu.async_copy(x_hbm.at[rowi.at[b]], gbuf.at[b], gsems.at[b])

    def chunk_out(i, b):
        pltpu.make_async_copy(
            x_hbm.at[rowi.at[b]], gbuf.at[b], gsems.at[b]).wait()

        def mul(g, _):
            val16 = mvbuf[b, pl.ds(g * 16, 16)]
            for k in range(16):
                e = g * 16 + k
                v = val16[k]
                for q in range(M // 16):
                    gbuf[b, e, pl.ds(q * 16, 16)] = \
                        gbuf[b, e, pl.ds(q * 16, 16)] * v
            return 0
        lax.fori_loop(0, CH // 16, mul, 0)
        pltpu.async_copy(
            gbuf.at[b], xash.at[coli.at[b]], ssems.at[b], add=True)

    for b in range(NBUF):
        eissue(0, b)

    def outer(i, _):
        @pl.when(i > 0)
        def _():
            for b in range(NBUF):
                pltpu.make_async_copy(
                    gbuf.at[b], xash.at[coli.at[b]], ssems.at[b]).wait()
        for b in range(NBUF):
            chunk_in(i, b)

        @pl.when(i < SCH // NBUF - 1)
        def _():
            for b in range(NBUF):
                eissue(i + 1, b)
        for b in range(NBUF):
            chunk_out(i, b)
        return 0
    lax.fori_loop(0, SCH // NBUF, outer, 0, unroll=False)

    for b in range(NBUF):
        pltpu.make_async_copy(
            gbuf.at[b], xash.at[coli.at[b]], ssems.at[b]).wait()
    plsc.subcore_barrier()

    pltpu.sync_copy(xash.at[pl.ds(s * ROWS, ROWS)],
                    xa_hbm.at[pl.ds(cbase + s * ROWS, ROWS)])


_spmm_kernel = functools.partial(
    pl.kernel,
    out_type=jax.ShapeDtypeStruct((NP, M), jnp.float32),
    mesh=_mesh,
    scratch_types=[
        pltpu.VMEM((NBUF, CH), jnp.int32),
        pltpu.VMEM((NBUF, CH), jnp.float32),
        pltpu.VMEM((NBUF, CH), jnp.float32),
        pltpu.VMEM((NBUF, CH), jnp.int32),
        pltpu.VMEM((NBUF, CH), jnp.int32),
        pltpu.VMEM((NBUF, CH, M), jnp.float32),
        pltpu.VMEM((64, M), jnp.float32),
        pltpu.VMEM_SHARED((NH, M), jnp.float32),
        pltpu.SemaphoreType.DMA((NBUF,)),
        pltpu.SemaphoreType.DMA((NBUF,)),
        pltpu.SemaphoreType.DMA((NBUF,)),
        pltpu.SemaphoreType.DMA((NBUF,)),
    ],
)(_spmm_body)


def _prep_body(x_ref, om0_ref, om1_ref, w0_ref, w1_ref, r2_ref,
               b0o, b1o, wt0, wt1):
    i = pl.program_id(0)
    xb = x_ref[...]
    dn = (((1,), (1,)), ((), ()))
    b0o[...] = lax.dot_general(xb, om0_ref[...], dn,
                               preferred_element_type=jnp.float32)
    b1o[...] = lax.dot_general(xb, om1_ref[...], dn,
                               preferred_element_type=jnp.float32)

    @pl.when(i == 0)
    def _():
        rho = jnp.sqrt(r2_ref[0, 0])
        kr = KAPPA / rho
        for w_ref, wt in ((w0_ref, wt0), (w1_ref, wt1)):
            w = w_ref[...]
            sa = jnp.sum(jnp.abs(w), axis=1)
            scale = jnp.where(sa > kr, kr / (sa + 1e-12), 1.0)
            wt[...] = (w * scale[:, None]).T


def _prep(x, Om0, Om1, W0, W1, r2):
    return pl.pallas_call(
        _prep_body,
        grid=(NP // BN,),
        in_specs=[
            pl.BlockSpec((BN, P), lambda i: (i, 0)),
            pl.BlockSpec((M, P), lambda i: (0, 0)),
            pl.BlockSpec((M, P), lambda i: (0, 0)),
            pl.BlockSpec((M, M), lambda i: (0, 0)),
            pl.BlockSpec((M, M), lambda i: (0, 0)),
            pl.BlockSpec((1, 16), lambda i: (0, 0)),
        ],
        out_specs=[
            pl.BlockSpec((BN, M), lambda i: (i, 0)),
            pl.BlockSpec((BN, M), lambda i: (i, 0)),
            pl.BlockSpec((M, M), lambda i: (0, 0)),
            pl.BlockSpec((M, M), lambda i: (0, 0)),
        ],
        out_shape=[
            jax.ShapeDtypeStruct((NP, M), jnp.float32),
            jax.ShapeDtypeStruct((NP, M), jnp.float32),
            jax.ShapeDtypeStruct((M, M), jnp.float32),
            jax.ShapeDtypeStruct((M, M), jnp.float32),
        ],
    )(x, Om0, Om1, W0, W1, r2)


def _step_body(xa_ref, wt_ref, b_ref, x_ref):
    acc = jnp.dot(xa_ref[...], wt_ref[...],
                  preferred_element_type=jnp.float32)
    x_ref[...] = jnp.maximum(acc + b_ref[...], 0.0)


def _step(xa, WnT, b):
    return pl.pallas_call(
        _step_body,
        grid=(NP // BN,),
        in_specs=[
            pl.BlockSpec((BN, M), lambda i: (i, 0)),
            pl.BlockSpec((M, M), lambda i: (0, 0)),
            pl.BlockSpec((BN, M), lambda i: (i, 0)),
        ],
        out_specs=pl.BlockSpec((BN, M), lambda i: (i, 0)),
        out_shape=jax.ShapeDtypeStruct((NP, M), jnp.float32),
    )(xa, WnT, b)


def _head_body(x_ref, hw_ref, hb_ref, o_ref):
    acc = jnp.dot(x_ref[...], hw_ref[...],
                  preferred_element_type=jnp.float32)
    o_ref[...] = acc + hb_ref[...]


def _head(x, head_W, head_b):
    return pl.pallas_call(
        _head_body,
        grid=(NP // BN,),
        in_specs=[
            pl.BlockSpec((BN, M), lambda i: (i, 0)),
            pl.BlockSpec((M, OUT), lambda i: (0, 0)),
            pl.BlockSpec((1, OUT), lambda i: (0, 0)),
        ],
        out_specs=pl.BlockSpec((BN, OUT), lambda i: (i, 0)),
        out_shape=jax.ShapeDtypeStruct((NP, OUT), jnp.float32),
    )(x, head_W, head_b)


def kernel(node_index, x, edge_index, adj_values, emb, W0, Om0, W1, Om1,
           head_W, head_b):
    row = edge_index[0].astype(jnp.int32)
    col = edge_index[1].astype(jnp.int32)
    rc = jnp.concatenate([
        (row << 14) | col,
        (jnp.arange(EPAD - E, dtype=jnp.int32) % N) * 16385,
    ]).reshape(NS * CHUNKS, CH)
    vals = jnp.concatenate([
        adj_values, jnp.zeros((EPAD - E,), jnp.float32)
    ]).reshape(NS * CHUNKS, CH)

    r2 = _power_kernel(rc, vals)

    rcflat = (row << 14) | col
    halfmask = (col >= NH).astype(jnp.int32)
    c1 = jnp.cumsum(halfmask)
    c0 = jnp.arange(1, E + 1, dtype=jnp.int32) - c1
    pos = jnp.where(halfmask == 0, c0 - 1, CAP + c1 - 1)
    pad_rc = (jnp.arange(2 * CAP, dtype=jnp.int32) % N) * 16385
    rc_p = pad_rc.at[pos].set(rcflat, unique_indices=True).reshape(-1, CH)
    vals_p = jnp.zeros((2 * CAP,), jnp.float32).at[pos].set(
        adj_values, unique_indices=True).reshape(-1, CH)

    xp = jnp.pad(x, ((0, NP - N), (0, 0)))
    B0, B1, Wt0, Wt1 = _prep(xp, Om0, Om1, W0, W1, r2.reshape(1, 16))

    X = emb
    X = jnp.pad(X, ((0, NP - N), (0, 0)))
    for Wt, B in ((Wt0, B0), (Wt1, B1)):
        for _ in range(MITR):
            xa = _spmm_kernel(rc_p, vals_p, X)
            X = _step(xa, Wt, B)
    return _head(X, head_W, head_b.reshape(1, OUT))[:N]


# speedup vs baseline: 1.3405x; 1.0521x over previous
import functools

import jax
import jax.numpy as jnp
from jax import lax
from jax.experimental import pallas as pl
from jax.experimental.pallas import tpu as pltpu
from jax.experimental.pallas import tpu_sc as plsc

N = 10000
E = 160000
P = 256
M = 128
OUT = 40
KAPPA = 0.9
MITR = 8
POWER_ITERS = 30

NC = 2
NS = 16
H = 64
CH = 128
CHUNKS = 80
EPT = CH * CHUNKS
EPAD = EPT * NS
NPAD = 10240
NP = 10240
BN = 2048

SCH = 44
CAP = NS * SCH * CH
NPADE = EPAD - E
OS = 2 * CAP + NPADE
FW = OS // NS

_mesh = plsc.VectorSubcoreMesh(
    core_axis_name="c", subcore_axis_name="s", num_cores=NC, num_subcores=NS)


_GDN = lax.GatherDimensionNumbers(
    offset_dims=(), collapsed_slice_dims=(0,), start_index_map=(0,))


def _dyngather16(v, idx):
    return lax.gather(v, idx[:, None], _GDN, slice_sizes=(1,),
                      mode=lax.GatherScatterMode.PROMISE_IN_BOUNDS)


def _vsum16(v):
    idx = lax.iota(jnp.int32, 16)
    for sh in (8, 4, 2, 1):
        v = v + _dyngather16(v, idx ^ sh)
    return v


def _power_body(rc_hbm, vals_hbm, pos_hbm, r2_hbm, rcp_hbm, valsp_hbm,
                rcv, valv, rowi, coli, vbuf, gath, prod, zv, obuf,
                posv, fbi, fbf, wa, wb, gsem, ssem, psem):
    c = lax.axis_index("c")
    s = lax.axis_index("s")
    stripe = NPAD // NS

    pltpu.sync_copy(rc_hbm.at[pl.ds(s * CHUNKS, CHUNKS)], rcv)
    pltpu.sync_copy(vals_hbm.at[pl.ds(s * CHUNKS, CHUNKS)], valv)

    def unpack(j, _):
        for q in range(CH // 16):
            rc16 = rcv[j, pl.ds(q * 16, 16)]
            rowi[j, pl.ds(q * 16, 16)] = lax.shift_right_logical(rc16, 14)
            coli[j, pl.ds(q * 16, 16)] = lax.bitwise_and(rc16, 16383)
        return 0
    lax.fori_loop(0, CHUNKS, unpack, 0)

    pltpu.sync_copy(pos_hbm.at[pl.ds(s * CHUNKS, CHUNKS)], posv)

    @pl.when(c == 0)
    def _():
        base = s * FW

        def fill(g, _):
            i16 = lax.iota(jnp.int32, 16) + (base + g * 16)
            fbi[pl.ds(g * 16, 16)] = \
                lax.bitwise_and(i16, 8191) * 16385
            fbf[pl.ds(g * 16, 16)] = jnp.zeros((16,), jnp.float32)
            return 0
        lax.fori_loop(0, FW // 16, fill, 0)
        pltpu.sync_copy(fbi, rcp_hbm.at[pl.ds(base, FW)])
        pltpu.sync_copy(fbf, valsp_hbm.at[pl.ds(base, FW)])
    plsc.subcore_barrier()

    @pl.when(c == 0)
    def _():
        def pscatter(j, _):
            pltpu.async_copy(rcv.at[j], rcp_hbm.at[posv.at[j]], psem)
            pltpu.async_copy(valv.at[j], valsp_hbm.at[posv.at[j]], psem)
            return 0
        lax.fori_loop(0, CHUNKS, pscatter, 0)

    c001 = jnp.full((16,), 0.01, jnp.float32)
    z16 = jnp.zeros((16,), jnp.float32)

    def init_c(g, _):
        zv[pl.ds(g * 16, 16)] = c001
        return 0
    lax.fori_loop(0, stripe // 16, init_c, 0)
    pltpu.sync_copy(zv, wa.at[pl.ds(s * stripe, stripe)])

    def init_z(g, _):
        zv[pl.ds(g * 16, 16)] = z16
        return 0
    lax.fori_loop(0, stripe // 16, init_z, 0)
    plsc.subcore_barrier()

    def half_iter(wcur, wnxt, carry):
        inv16, ss_prev, ss_cur = carry

        pltpu.sync_copy(zv, wnxt.at[pl.ds(s * stripe, stripe)])
        plsc.subcore_barrier()

        def gissue(j, _):
            pltpu.async_copy(wcur.at[coli.at[j]], gath.at[j], gsem)
            return 0
        lax.fori_loop(0, CHUNKS, gissue, 0)

        def gdrain(j, _):
            pltpu.make_async_copy(
                wcur.at[coli.at[0]], gath.at[0], gsem).wait()
            return 0
        lax.fori_loop(0, CHUNKS, gdrain, 0)

        def pcompute(j, _):
            for q in range(CH // 16):
                prod[j, pl.ds(q * 16, 16)] = \
                    valv[j, pl.ds(q * 16, 16)] * \
                    (gath[j, pl.ds(q * 16, 16)] * inv16)
            pltpu.async_copy(prod.at[j], wnxt.at[rowi.at[j]], ssem, add=True)
            return 0
        lax.fori_loop(0, CHUNKS, pcompute, 0)

        def sdrain(j, _):
            pltpu.make_async_copy(
                prod.at[0], wnxt.at[rowi.at[0]], ssem).wait()
            return 0
        lax.fori_loop(0, CHUNKS, sdrain, 0)
        plsc.subcore_barrier()

        pltpu.sync_copy(wnxt.at[pl.ds(0, N)], vbuf.at[pl.ds(0, N)])

        acc = jnp.zeros((16,), jnp.float32)

        def ssbody(g, a):
            w16 = vbuf[pl.ds(g * 16, 16)]
            return a + w16 * w16
        acc = lax.fori_loop(0, N // 16, ssbody, acc)
        ss = _vsum16(acc)
        plsc.subcore_barrier()
        return (1.0 / ss, ss_cur, ss)

    def iter_pair(_, carry):
        carry = half_iter(wa, wb, carry)
        carry = half_iter(wb, wa, carry)
        return carry

    one = jnp.ones((16,), jnp.float32)
    _, ss_prev, ss_cur = lax.fori_loop(
        0, POWER_ITERS // 2, iter_pair, (one, one, one))

    r2 = ss_prev * ss_cur

    @pl.when(c == 0)
    def _():
        def pdrain(j, _):
            pltpu.make_async_copy(
                rcv.at[0], rcp_hbm.at[posv.at[0]], psem).wait()
            pltpu.make_async_copy(
                valv.at[0], valsp_hbm.at[posv.at[0]], psem).wait()
            return 0
        lax.fori_loop(0, CHUNKS, pdrain, 0)

    @pl.when(jnp.logical_and(c == 0, s == 0))
    def _():
        obuf[...] = r2
        pltpu.sync_copy(obuf, r2_hbm)


_power_kernel = functools.partial(
    pl.kernel,
    out_type=(jax.ShapeDtypeStruct((16,), jnp.float32),
              jax.ShapeDtypeStruct((OS,), jnp.int32),
              jax.ShapeDtypeStruct((OS,), jnp.float32)),
    mesh=_mesh,
    scratch_types=[
        pltpu.VMEM((CHUNKS, CH), jnp.int32),
        pltpu.VMEM((CHUNKS, CH), jnp.float32),
        pltpu.VMEM((CHUNKS, CH), jnp.int32),
        pltpu.VMEM((CHUNKS, CH), jnp.int32),
        pltpu.VMEM((NPAD,), jnp.float32),
        pltpu.VMEM((CHUNKS, CH), jnp.float32),
        pltpu.VMEM((CHUNKS, CH), jnp.float32),
        pltpu.VMEM((NPAD // NS,), jnp.float32),
        pltpu.VMEM((16,), jnp.float32),
        pltpu.VMEM((CHUNKS, CH), jnp.int32),
        pltpu.VMEM((FW,), jnp.int32),
        pltpu.VMEM((FW,), jnp.float32),
        pltpu.VMEM_SHARED((NPAD,), jnp.float32),
        pltpu.VMEM_SHARED((NPAD,), jnp.float32),
        pltpu.SemaphoreType.DMA,
        pltpu.SemaphoreType.DMA,
        pltpu.SemaphoreType.DMA,
    ],
)(_power_body)


NBUF = 2
NH = NP // NC
ROWS = NH // NS

def _spmm_body(rc_hbm, vals_hbm, x_hbm, xa_hbm,
               rcbuf, valbuf, mvbuf, rowi, coli, gbuf, zbuf, xash,
               esems, vsems, gsems, ssems):
    c = lax.axis_index("c")
    s = lax.axis_index("s")
    cbase = c * NH

    z16 = jnp.zeros((16,), jnp.float32)

    def zinit(e, _):
        for q in range(M // 16):
            zbuf[e, pl.ds(q * 16, 16)] = z16
        return 0
    lax.fori_loop(0, 64, zinit, 0)
    for k in range(ROWS // 64):
        pltpu.sync_copy(zbuf, xash.at[pl.ds(s * ROWS + k * 64, 64)])

    plsc.subcore_barrier()

    def eissue(i, b):
        jr = (((c * NS + s) * SCH) + i * NBUF + b) * CH
        pltpu.async_copy(rc_hbm.at[pl.ds(jr, CH)], rcbuf.at[b],
                         esems.at[b])
        pltpu.async_copy(vals_hbm.at[pl.ds(jr, CH)], valbuf.at[b],
                         vsems.at[b])

    def chunk_in(i, b):
        pltpu.make_async_copy(
            rc_hbm.at[pl.ds(0, CH)], rcbuf.at[b], esems.at[b]).wait()
        pltpu.make_async_copy(
            vals_hbm.at[pl.ds(0, CH)], valbuf.at[b], vsems.at[b]).wait()
        for q in range(CH // 16):
            rc16 = rcbuf[b, pl.ds(q * 16, 16)]
            row16 = lax.shift_right_logical(rc16, 14)
            col16 = lax.bitwise_and(rc16, 16383)
            lcol = col16 - cbase
            inhalf = jnp.logical_and(lcol >= 0, lcol < NH)
            rowi[b, pl.ds(q * 16, 16)] = row16
            coli[b, pl.ds(q * 16, 16)] = jnp.clip(lcol, 0, NH - 1)
            mvbuf[b, pl.ds(q * 16, 16)] = jnp.where(
                inhalf, valbuf[b, pl.ds(q * 16, 16)], 0.0)
        pltpu.async_copy(x_hbm.at[rowi.at[b]], gbuf.at[b], gsems.at[b])

    def chunk_out(i, b):
        pltpu.make_async_copy(
            x_hbm.at[rowi.at[b]], gbuf.at[b], gsems.at[b]).wait()

        def mul(g, _):
            val16 = mvbuf[b, pl.ds(g * 16, 16)]
            for k in range(16):
                e = g * 16 + k
                v = val16[k]
                for q in range(M // 16):
                    gbuf[b, e, pl.ds(q * 16, 16)] = \
                        gbuf[b, e, pl.ds(q * 16, 16)] * v
            return 0
        lax.fori_loop(0, CH // 16, mul, 0)
        pltpu.async_copy(
            gbuf.at[b], xash.at[coli.at[b]], ssems.at[b], add=True)

    for b in range(NBUF):
        eissue(0, b)

    def outer(i, _):
        @pl.when(i > 0)
        def _():
            for b in range(NBUF):
                pltpu.make_async_copy(
                    gbuf.at[b], xash.at[coli.at[b]], ssems.at[b]).wait()
        for b in range(NBUF):
            chunk_in(i, b)

        @pl.when(i < SCH // NBUF - 1)
        def _():
            for b in range(NBUF):
                eissue(i + 1, b)
        for b in range(NBUF):
            chunk_out(i, b)
        return 0
    lax.fori_loop(0, SCH // NBUF, outer, 0, unroll=False)

    for b in range(NBUF):
        pltpu.make_async_copy(
            gbuf.at[b], xash.at[coli.at[b]], ssems.at[b]).wait()
    plsc.subcore_barrier()

    pltpu.sync_copy(xash.at[pl.ds(s * ROWS, ROWS)],
                    xa_hbm.at[pl.ds(cbase + s * ROWS, ROWS)])


_spmm_kernel = functools.partial(
    pl.kernel,
    out_type=jax.ShapeDtypeStruct((NP, M), jnp.float32),
    mesh=_mesh,
    scratch_types=[
        pltpu.VMEM((NBUF, CH), jnp.int32),
        pltpu.VMEM((NBUF, CH), jnp.float32),
        pltpu.VMEM((NBUF, CH), jnp.float32),
        pltpu.VMEM((NBUF, CH), jnp.int32),
        pltpu.VMEM((NBUF, CH), jnp.int32),
        pltpu.VMEM((NBUF, CH, M), jnp.float32),
        pltpu.VMEM((64, M), jnp.float32),
        pltpu.VMEM_SHARED((NH, M), jnp.float32),
        pltpu.SemaphoreType.DMA((NBUF,)),
        pltpu.SemaphoreType.DMA((NBUF,)),
        pltpu.SemaphoreType.DMA((NBUF,)),
        pltpu.SemaphoreType.DMA((NBUF,)),
    ],
)(_spmm_body)


def _prep_body(x_ref, om0_ref, om1_ref, w0_ref, w1_ref, r2_ref,
               b0o, b1o, wt0, wt1):
    i = pl.program_id(0)
    xb = x_ref[...]
    dn = (((1,), (1,)), ((), ()))
    b0o[...] = lax.dot_general(xb, om0_ref[...], dn,
                               preferred_element_type=jnp.float32)
    b1o[...] = lax.dot_general(xb, om1_ref[...], dn,
                               preferred_element_type=jnp.float32)

    @pl.when(i == 0)
    def _():
        rho = jnp.sqrt(r2_ref[0, 0])
        kr = KAPPA / rho
        for w_ref, wt in ((w0_ref, wt0), (w1_ref, wt1)):
            w = w_ref[...]
            sa = jnp.sum(jnp.abs(w), axis=1)
            scale = jnp.where(sa > kr, kr / (sa + 1e-12), 1.0)
            wt[...] = (w * scale[:, None]).T


def _prep(x, Om0, Om1, W0, W1, r2):
    return pl.pallas_call(
        _prep_body,
        grid=(NP // BN,),
        in_specs=[
            pl.BlockSpec((BN, P), lambda i: (i, 0)),
            pl.BlockSpec((M, P), lambda i: (0, 0)),
            pl.BlockSpec((M, P), lambda i: (0, 0)),
            pl.BlockSpec((M, M), lambda i: (0, 0)),
            pl.BlockSpec((M, M), lambda i: (0, 0)),
            pl.BlockSpec((1, 16), lambda i: (0, 0)),
        ],
        out_specs=[
            pl.BlockSpec((BN, M), lambda i: (i, 0)),
            pl.BlockSpec((BN, M), lambda i: (i, 0)),
            pl.BlockSpec((M, M), lambda i: (0, 0)),
            pl.BlockSpec((M, M), lambda i: (0, 0)),
        ],
        out_shape=[
            jax.ShapeDtypeStruct((NP, M), jnp.float32),
            jax.ShapeDtypeStruct((NP, M), jnp.float32),
            jax.ShapeDtypeStruct((M, M), jnp.float32),
            jax.ShapeDtypeStruct((M, M), jnp.float32),
        ],
    )(x, Om0, Om1, W0, W1, r2)


def _step_body(xa_ref, wt_ref, b_ref, x_ref):
    acc = jnp.dot(xa_ref[...], wt_ref[...],
                  preferred_element_type=jnp.float32)
    x_ref[...] = jnp.maximum(acc + b_ref[...], 0.0)


def _step(xa, WnT, b):
    return pl.pallas_call(
        _step_body,
        grid=(NP // BN,),
        in_specs=[
            pl.BlockSpec((BN, M), lambda i: (i, 0)),
            pl.BlockSpec((M, M), lambda i: (0, 0)),
            pl.BlockSpec((BN, M), lambda i: (i, 0)),
        ],
        out_specs=pl.BlockSpec((BN, M), lambda i: (i, 0)),
        out_shape=jax.ShapeDtypeStruct((NP, M), jnp.float32),
    )(xa, WnT, b)


def _head_body(x_ref, hw_ref, hb_ref, o_ref):
    acc = jnp.dot(x_ref[...], hw_ref[...],
                  preferred_element_type=jnp.float32)
    o_ref[...] = acc + hb_ref[...]


def _head(x, head_W, head_b):
    return pl.pallas_call(
        _head_body,
        grid=(NP // BN,),
        in_specs=[
            pl.BlockSpec((BN, M), lambda i: (i, 0)),
            pl.BlockSpec((M, OUT), lambda i: (0, 0)),
            pl.BlockSpec((1, OUT), lambda i: (0, 0)),
        ],
        out_specs=pl.BlockSpec((BN, OUT), lambda i: (i, 0)),
        out_shape=jax.ShapeDtypeStruct((NP, OUT), jnp.float32),
    )(x, head_W, head_b)


def kernel(node_index, x, edge_index, adj_values, emb, W0, Om0, W1, Om1,
           head_W, head_b):
    row = edge_index[0].astype(jnp.int32)
    col = edge_index[1].astype(jnp.int32)
    rc = jnp.concatenate([
        (row << 14) | col,
        (jnp.arange(EPAD - E, dtype=jnp.int32) % N) * 16385,
    ]).reshape(NS * CHUNKS, CH)
    vals = jnp.concatenate([
        adj_values, jnp.zeros((EPAD - E,), jnp.float32)
    ]).reshape(NS * CHUNKS, CH)

    halfmask = (col >= NH).astype(jnp.int32)
    c1 = jnp.cumsum(halfmask)
    c0 = jnp.arange(1, E + 1, dtype=jnp.int32) - c1
    pos = jnp.where(halfmask == 0, c0 - 1, CAP + c1 - 1)
    pos = jnp.concatenate([
        pos, 2 * CAP + jnp.arange(NPADE, dtype=jnp.int32)
    ]).reshape(NS * CHUNKS, CH)

    r2, rc_p, vals_p = _power_kernel(rc, vals, pos)

    xp = jnp.pad(x, ((0, NP - N), (0, 0)))
    B0, B1, Wt0, Wt1 = _prep(xp, Om0, Om1, W0, W1, r2.reshape(1, 16))

    X = emb
    X = jnp.pad(X, ((0, NP - N), (0, 0)))
    for Wt, B in ((Wt0, B0), (Wt1, B1)):
        for _ in range(MITR):
            xa = _spmm_kernel(rc_p, vals_p, X)
            X = _step(xa, Wt, B)
    return _head(X, head_W, head_b.reshape(1, OUT))[:N]
